# baseline (speedup 1.0000x reference)
import functools

import jax
import jax.numpy as jnp
from jax import lax
from jax.experimental import pallas as pl
from jax.experimental.pallas import tpu as pltpu
from jax.experimental.pallas import tpu_sc as plsc

H = 64
W = 64
D = 64
Q = W * D
V = H * Q
NSEG_PAD = 262400
T = 2 * NSEG_PAD
STRIPE = T // 16
ZB = STRIPE // 2
NENT = 2 * 4 * V
EPT = (2 * V) // 16
CHUNK = 8192
NCHUNK = EPT // CHUNK


def _prop_kernel(y_ref, yp_ref, ids_ref, vals_ref):
    bc = pl.program_id(0)
    c = bc % 2 + 1
    yk = y_ref[0]
    mask = yk == c
    hi = lax.broadcasted_iota(jnp.int32, (H, Q), 0)
    qi = lax.broadcasted_iota(jnp.int32, (H, Q), 1)
    di = qi % D
    lin = hi * Q + qi + 1
    zero = jnp.zeros((H, Q), jnp.int32)
    not_h0 = hi != 0
    not_h63 = hi != (H - 1)
    not_w0 = qi >= D
    not_w63 = qi < Q - D
    not_d0 = di != 0
    not_d63 = di != (D - 1)

    def mneigh(x):
        m = jnp.maximum(x, jnp.where(not_h0, pltpu.roll(x, 1, 0), zero))
        m = jnp.maximum(m, jnp.where(not_h63, pltpu.roll(x, H - 1, 0), zero))
        m = jnp.maximum(m, jnp.where(not_w0, pltpu.roll(x, D, 1), zero))
        m = jnp.maximum(m, jnp.where(not_w63, pltpu.roll(x, Q - D, 1), zero))
        m = jnp.maximum(m, jnp.where(not_d0, pltpu.roll(x, 1, 1), zero))
        m = jnp.maximum(m, jnp.where(not_d63, pltpu.roll(x, Q - 1, 1), zero))
        return m

    labels = jnp.where(mask, lin, 0)
    labels = lax.fori_loop(
        0, 8, lambda i, l: jnp.where(mask, mneigh(l), l), labels)
    vor = lax.fori_loop(
        0, 8, lambda i, v: jnp.where(v > 0, v, mneigh(v)), labels)

    p = jnp.maximum(jax.nn.sigmoid(yp_ref[0, 0]), 1e-20)
    g = mask.astype(jnp.float32)
    ids_ref[0, 0] = vor
    ids_ref[0, 1] = vor + NSEG_PAD
    vals_ref[0, 0] = p * g
    vals_ref[0, 1] = p + g


_prop_call = pl.pallas_call(
    _prop_kernel,
    grid=(4,),
    in_specs=[
        pl.BlockSpec((1, H, Q), lambda bc: (bc // 2, 0, 0)),
        pl.BlockSpec((1, 1, H, Q), lambda bc: (bc // 2, bc % 2, 0, 0)),
    ],
    out_specs=[
        pl.BlockSpec((1, 2, H, Q), lambda bc: (bc, 0, 0, 0)),
        pl.BlockSpec((1, 2, H, Q), lambda bc: (bc, 0, 0, 0)),
    ],
    out_shape=[
        jax.ShapeDtypeStruct((4, 2, H, Q), jnp.int32),
        jax.ShapeDtypeStruct((4, 2, H, Q), jnp.float32),
    ],
)


def _sc_scatter_body(ids_hbm, vals_hbm, out_hbm,
                     idsv0, valsv0, idsv1, valsv1, zbuf, obuf, table,
                     sem0, sem1, sem2, sem3):
    cid = lax.axis_index("c")
    sid = lax.axis_index("s")
    bufs = ((idsv0, valsv0, sem0, sem1), (idsv1, valsv1, sem2, sem3))

    def zfill(i, carry):
        zbuf[pl.ds(i * 16, 16)] = jnp.zeros((16,), jnp.float32)
        return carry

    lax.fori_loop(0, ZB // 16, zfill, 0)

    def ent0(phase, k):
        slab = 2 * cid + phase
        return slab * (2 * V) + sid * EPT + k * CHUNK

    def start_load(e0, b):
        idsb, valsb, s1, s2 = b
        h1 = pltpu.make_async_copy(ids_hbm.at[pl.ds(e0, CHUNK)], idsb, s1)
        h2 = pltpu.make_async_copy(vals_hbm.at[pl.ds(e0, CHUNK)], valsb, s2)
        h1.start()
        h2.start()
        return (h1, h2)

    pending = start_load(ent0(0, 0), bufs[0])
    for phase in range(2):
        slab = 2 * cid + phase
        for k in range(2):
            pltpu.sync_copy(zbuf, table.at[pl.ds(sid * STRIPE + k * ZB, ZB)])
        plsc.subcore_barrier()

        for k in range(NCHUNK):
            idsb, valsb, _, _ = bufs[k % 2]
            for h in pending:
                h.wait()
            if k + 1 < NCHUNK:
                pending = start_load(ent0(phase, k + 1), bufs[(k + 1) % 2])
            elif phase == 0:
                pending = start_load(ent0(1, 0), bufs[0])
            pltpu.sync_copy(valsb, table.at[idsb], add=True)
        plsc.subcore_barrier()
        for k in range(2):
            off = sid * STRIPE + k * ZB
            pltpu.sync_copy(table.at[pl.ds(off, ZB)], obuf)
            pltpu.sync_copy(obuf, out_hbm.at[pl.ds(slab * T + off, ZB)])


@functools.cache
def _sc_scatter():
    return pl.kernel(
        _sc_scatter_body,
        mesh=plsc.VectorSubcoreMesh(core_axis_name="c", subcore_axis_name="s"),
        out_type=jax.ShapeDtypeStruct((4 * T,), jnp.float32),
        scratch_types=[
            pltpu.VMEM((CHUNK,), jnp.int32),
            pltpu.VMEM((CHUNK,), jnp.float32),
            pltpu.VMEM((CHUNK,), jnp.int32),
            pltpu.VMEM((CHUNK,), jnp.float32),
            pltpu.VMEM((ZB,), jnp.float32),
            pltpu.VMEM((ZB,), jnp.float32),
            pltpu.VMEM_SHARED((T,), jnp.float32),
            pltpu.SemaphoreType.DMA,
            pltpu.SemaphoreType.DMA,
            pltpu.SemaphoreType.DMA,
            pltpu.SemaphoreType.DMA,
        ],
    )


def _reduce_kernel(tab_ref, out_ref):
    eps = jnp.float32(1e-5)
    acc = jnp.float32(0.0)
    for s in range(4):
        inter = tab_ref[2 * s : 2 * s + 1, :]
        denom = tab_ref[2 * s + 1 : 2 * s + 2, :]
        valid = (denom > 0).astype(jnp.float32)
        dice = (2.0 * inter + eps) / (denom + eps)
        num = jnp.sum(dice * valid)
        nval = jnp.sum(valid)
        acc = acc + num / jnp.maximum(nval, 1.0)
    out_ref[:, :] = jnp.broadcast_to(1.0 - acc * 0.25, (1, 1))


_reduce_call = pl.pallas_call(
    _reduce_kernel,
    out_shape=jax.ShapeDtypeStruct((1, 1), jnp.float32),
)


def kernel(y_pred, y):
    y2 = y[:, 0].reshape(2, H, Q)
    yp2 = y_pred[:, 1:].reshape(2, 2, H, Q)
    ids, vals = _prop_call(y2, yp2)
    tab = _sc_scatter()(ids.reshape(NENT), vals.reshape(NENT))
    out = _reduce_call(tab.reshape(8, NSEG_PAD))
    return out[0, 0]


# speedup vs baseline: 1.3739x; 1.3739x over previous
import functools

import jax
import jax.numpy as jnp
from jax import lax
from jax.experimental import pallas as pl
from jax.experimental.pallas import tpu as pltpu
from jax.experimental.pallas import tpu_sc as plsc

H = 64
W = 64
D = 64
L = 2 * D
V = H * W * D
PK = W * L
NSEG_PAD = 262400
T = 2 * NSEG_PAD
STRIPE = T // 16
ZB = STRIPE // 2
NIDS = 2 * 2 * V
NENT = 2 * NIDS
IPT = NIDS // 32
CHUNK = 8192
NCHUNK = IPT // CHUNK


def _prop_kernel(y_ref, yp_ref, ids_ref, vals_ref):
    yk = y_ref[0]
    ypair = jnp.concatenate([yk, yk], axis=-1)
    li = lax.broadcasted_iota(jnp.int32, (H, W, L), 2)
    cvec = 1 + (li >= D).astype(jnp.int32)
    mask = ypair == cvec
    hi = lax.broadcasted_iota(jnp.int32, (H, W, L), 0)
    wi = lax.broadcasted_iota(jnp.int32, (H, W, L), 1)
    di = li % D
    lin = hi * (W * D) + wi * D + di + 1
    zero = jnp.zeros((H, W, L), jnp.int32)
    not_d0 = di != 0
    not_d63 = di != (D - 1)

    def mneigh(x):
        zh = jnp.zeros((1, W, L), x.dtype)
        m = jnp.maximum(x, jnp.concatenate([zh, x[:-1]], axis=0))
        m = jnp.maximum(m, jnp.concatenate([x[1:], zh], axis=0))
        zw = jnp.zeros((H, 1, L), x.dtype)
        m = jnp.maximum(m, jnp.concatenate([zw, x[:, :-1]], axis=1))
        m = jnp.maximum(m, jnp.concatenate([x[:, 1:], zw], axis=1))
        m = jnp.maximum(m, jnp.where(not_d0, pltpu.roll(x, 1, 2), zero))
        m = jnp.maximum(m, jnp.where(not_d63, pltpu.roll(x, L - 1, 2), zero))
        return m

    labels = jnp.where(mask, lin, 0)
    labels = lax.fori_loop(
        0, 8, lambda i, l: jnp.where(mask, mneigh(l), l), labels)
    vor = lax.fori_loop(
        0, 8, lambda i, v: jnp.where(v > 0, v, mneigh(v)), labels)

    ypp = jnp.concatenate([yp_ref[0, 0], yp_ref[0, 1]], axis=-1)
    p = jnp.maximum(jax.nn.sigmoid(ypp), 1e-20)
    g = mask.astype(jnp.float32)
    ids_ref[0] = vor + (cvec - 1) * NSEG_PAD
    vals_ref[0, 0] = p * g
    vals_ref[0, 1] = p + g


_prop_call = pl.pallas_call(
    _prop_kernel,
    grid=(2,),
    in_specs=[
        pl.BlockSpec((1, H, W, D), lambda b: (b, 0, 0, 0)),
        pl.BlockSpec((1, 2, H, W, D), lambda b: (b, 0, 0, 0, 0)),
    ],
    out_specs=[
        pl.BlockSpec((1, H, W, L), lambda b: (b, 0, 0, 0)),
        pl.BlockSpec((1, 2, H, W, L), lambda b: (b, 0, 0, 0, 0)),
    ],
    out_shape=[
        jax.ShapeDtypeStruct((2, H, W, L), jnp.int32),
        jax.ShapeDtypeStruct((2, 2, H, W, L), jnp.float32),
    ],
)


def _sc_scatter_body(ids_hbm, vals_hbm, out_hbm,
                     idsb0, idsb1, idsb2, idsb3, valsv0, valsv1,
                     zbuf, obuf, table,
                     isem0, isem1, isem2, isem3, vsem0, vsem1):
    cid = lax.axis_index("c")
    sid = lax.axis_index("s")
    idsbufs = (idsb0, idsb1, idsb2, idsb3)
    vbufs = ((valsv0, vsem0), (valsv1, vsem1))

    ids_base = cid * (NIDS // 2) + sid * IPT
    ih = []
    for k, (ib, isem) in enumerate(zip(idsbufs, (isem0, isem1, isem2, isem3))):
        h = pltpu.make_async_copy(
            ids_hbm.at[pl.ds(ids_base + k * CHUNK, CHUNK)], ib, isem)
        h.start()
        ih.append(h)

    def vals_off(phase, k):
        return (cid * (NENT // 2) + phase * NIDS + sid * IPT + k * CHUNK)

    def start_vload(e0, vb):
        valsb, s = vb
        h = pltpu.make_async_copy(vals_hbm.at[pl.ds(e0, CHUNK)], valsb, s)
        h.start()
        return h

    vpend = start_vload(vals_off(0, 0), vbufs[0])

    def zfill(i, carry):
        zbuf[pl.ds(i * 16, 16)] = jnp.zeros((16,), jnp.float32)
        return carry

    lax.fori_loop(0, ZB // 16, zfill, 0)
    for k in range(2):
        pltpu.sync_copy(zbuf, table.at[pl.ds(sid * STRIPE + k * ZB, ZB)])
    plsc.subcore_barrier()

    for phase in range(2):
        for k in range(NCHUNK):
            valsb, _ = vbufs[k % 2]
            vpend.wait()
            if k + 1 < NCHUNK:
                vpend = start_vload(vals_off(phase, k + 1), vbufs[(k + 1) % 2])
            elif phase == 0:
                vpend = start_vload(vals_off(1, 0), vbufs[0])
            if phase == 0:
                ih[k].wait()
            pltpu.sync_copy(valsb, table.at[idsbufs[k]], add=True)
        plsc.subcore_barrier()
        for k in range(2):
            off = sid * STRIPE + k * ZB
            pltpu.sync_copy(table.at[pl.ds(off, ZB)], obuf)
            pltpu.sync_copy(
                obuf, out_hbm.at[pl.ds(cid * (2 * T) + phase * T + off, ZB)])
        if phase == 0:
            for k in range(2):
                pltpu.sync_copy(
                    zbuf, table.at[pl.ds(sid * STRIPE + k * ZB, ZB)])
            plsc.subcore_barrier()


@functools.cache
def _sc_scatter():
    return pl.kernel(
        _sc_scatter_body,
        mesh=plsc.VectorSubcoreMesh(core_axis_name="c", subcore_axis_name="s"),
        out_type=jax.ShapeDtypeStruct((4 * T,), jnp.float32),
        scratch_types=[
            pltpu.VMEM((CHUNK,), jnp.int32),
            pltpu.VMEM((CHUNK,), jnp.int32),
            pltpu.VMEM((CHUNK,), jnp.int32),
            pltpu.VMEM((CHUNK,), jnp.int32),
            pltpu.VMEM((CHUNK,), jnp.float32),
            pltpu.VMEM((CHUNK,), jnp.float32),
            pltpu.VMEM((ZB,), jnp.float32),
            pltpu.VMEM((ZB,), jnp.float32),
            pltpu.VMEM_SHARED((T,), jnp.float32),
            pltpu.SemaphoreType.DMA,
            pltpu.SemaphoreType.DMA,
            pltpu.SemaphoreType.DMA,
            pltpu.SemaphoreType.DMA,
            pltpu.SemaphoreType.DMA,
            pltpu.SemaphoreType.DMA,
        ],
    )


def _reduce_kernel(tab_ref, out_ref):
    eps = jnp.float32(1e-5)
    acc = jnp.float32(0.0)
    for s in range(4):
        b, c = s // 2, s % 2
        inter = tab_ref[4 * b + c : 4 * b + c + 1, :]
        denom = tab_ref[4 * b + 2 + c : 4 * b + 3 + c, :]
        valid = (denom > 0).astype(jnp.float32)
        dice = (2.0 * inter + eps) / (denom + eps)
        num = jnp.sum(dice * valid)
        nval = jnp.sum(valid)
        acc = acc + num / jnp.maximum(nval, 1.0)
    out_ref[:, :] = jnp.broadcast_to(1.0 - acc * 0.25, (1, 1))


_reduce_call = pl.pallas_call(
    _reduce_kernel,
    out_shape=jax.ShapeDtypeStruct((1, 1), jnp.float32),
)


def kernel(y_pred, y):
    y2 = y[:, 0]
    yp2 = y_pred[:, 1:]
    ids, vals = _prop_call(y2, yp2)
    tab = _sc_scatter()(ids.reshape(NIDS), vals.reshape(NENT))
    out = _reduce_call(tab.reshape(8, NSEG_PAD))
    return out[0, 0]


# speedup vs baseline: 1.3739x; 1.0000x over previous
import functools

import jax
import jax.numpy as jnp
from jax import lax
from jax.experimental import pallas as pl
from jax.experimental.pallas import tpu as pltpu
from jax.experimental.pallas import tpu_sc as plsc

H = 64
W = 64
D = 64
L = 2 * D
V = H * W * D
PK = W * L
NSEG_PAD = 262400
T = 2 * NSEG_PAD
STRIPE = T // 16
ZB = STRIPE // 2
NIDS = 2 * 2 * V
NENT = 2 * NIDS
IPT = NIDS // 32
CHUNK = 8192
NCHUNK = IPT // CHUNK


def _prop_kernel(y_ref, yp_ref, ids_ref, vals_ref):
    yk = y_ref[0]
    ypair = jnp.concatenate([yk, yk], axis=-1)
    li = lax.broadcasted_iota(jnp.int32, (H, W, L), 2)
    cvec = 1 + (li >= D).astype(jnp.int32)
    mask = ypair == cvec
    hi = lax.broadcasted_iota(jnp.int32, (H, W, L), 0)
    wi = lax.broadcasted_iota(jnp.int32, (H, W, L), 1)
    di = li % D
    lin = hi * (W * D) + wi * D + di + 1
    zero = jnp.zeros((H, W, L), jnp.int32)
    not_d0 = di != 0
    not_d63 = di != (D - 1)

    def mneigh(x):
        zh = jnp.zeros((1, W, L), x.dtype)
        m = jnp.maximum(x, jnp.concatenate([zh, x[:-1]], axis=0))
        m = jnp.maximum(m, jnp.concatenate([x[1:], zh], axis=0))
        zw = jnp.zeros((H, 1, L), x.dtype)
        m = jnp.maximum(m, jnp.concatenate([zw, x[:, :-1]], axis=1))
        m = jnp.maximum(m, jnp.concatenate([x[:, 1:], zw], axis=1))
        m = jnp.maximum(m, jnp.where(not_d0, pltpu.roll(x, 1, 2), zero))
        m = jnp.maximum(m, jnp.where(not_d63, pltpu.roll(x, L - 1, 2), zero))
        return m

    labels = jnp.where(mask, lin, 0)
    labels = lax.fori_loop(
        0, 8, lambda i, l: jnp.where(mask, mneigh(l), l), labels)
    vor = lax.fori_loop(
        0, 8, lambda i, v: jnp.where(v > 0, v, mneigh(v)), labels)

    ypp = jnp.concatenate([yp_ref[0, 0], yp_ref[0, 1]], axis=-1)
    p = jnp.maximum(jax.nn.sigmoid(ypp), 1e-20)
    g = mask.astype(jnp.float32)
    ids_ref[0] = vor + (cvec - 1) * NSEG_PAD
    vals_ref[0, 0] = p * g
    vals_ref[0, 1] = p + g


_prop_call = pl.pallas_call(
    _prop_kernel,
    grid=(2,),
    in_specs=[
        pl.BlockSpec((1, H, W, D), lambda b: (b, 0, 0, 0)),
        pl.BlockSpec((1, 2, H, W, D), lambda b: (b, 0, 0, 0, 0)),
    ],
    out_specs=[
        pl.BlockSpec((1, H, W, L), lambda b: (b, 0, 0, 0)),
        pl.BlockSpec((1, 2, H, W, L), lambda b: (b, 0, 0, 0, 0)),
    ],
    out_shape=[
        jax.ShapeDtypeStruct((2, H, W, L), jnp.int32),
        jax.ShapeDtypeStruct((2, 2, H, W, L), jnp.float32),
    ],
)


def _sc_scatter_body(ids_hbm, vals_hbm, out_hbm,
                     idsb0, idsb1, idsb2, idsb3, valsv0, valsv1,
                     zbuf, obuf, table,
                     isem0, isem1, isem2, isem3, vsem0, vsem1):
    cid = lax.axis_index("c")
    sid = lax.axis_index("s")
    idsbufs = (idsb0, idsb1, idsb2, idsb3)
    vbufs = ((valsv0, vsem0), (valsv1, vsem1))

    ids_base = cid * (NIDS // 2) + sid * IPT
    ih = []
    for k, (ib, isem) in enumerate(zip(idsbufs, (isem0, isem1, isem2, isem3))):
        h = pltpu.make_async_copy(
            ids_hbm.at[pl.ds(ids_base + k * CHUNK, CHUNK)], ib, isem)
        h.start()
        ih.append(h)

    def vals_off(phase, k):
        return (cid * (NENT // 2) + phase * (NIDS // 2) + sid * IPT + k * CHUNK)

    def start_vload(e0, vb):
        valsb, s = vb
        h = pltpu.make_async_copy(vals_hbm.at[pl.ds(e0, CHUNK)], valsb, s)
        h.start()
        return h

    vpend = start_vload(vals_off(0, 0), vbufs[0])

    def zfill(i, carry):
        zbuf[pl.ds(i * 16, 16)] = jnp.zeros((16,), jnp.float32)
        return carry

    lax.fori_loop(0, ZB // 16, zfill, 0)
    for k in range(2):
        pltpu.sync_copy(zbuf, table.at[pl.ds(sid * STRIPE + k * ZB, ZB)])
    plsc.subcore_barrier()

    for phase in range(2):
        for k in range(NCHUNK):
            valsb, _ = vbufs[k % 2]
            vpend.wait()
            if k + 1 < NCHUNK:
                vpend = start_vload(vals_off(phase, k + 1), vbufs[(k + 1) % 2])
            elif phase == 0:
                vpend = start_vload(vals_off(1, 0), vbufs[0])
            if phase == 0:
                ih[k].wait()
            pltpu.sync_copy(valsb, table.at[idsbufs[k]], add=True)
        plsc.subcore_barrier()
        for k in range(2):
            off = sid * STRIPE + k * ZB
            pltpu.sync_copy(table.at[pl.ds(off, ZB)], obuf)
            pltpu.sync_copy(
                obuf, out_hbm.at[pl.ds(cid * (2 * T) + phase * T + off, ZB)])
        if phase == 0:
            for k in range(2):
                pltpu.sync_copy(
                    zbuf, table.at[pl.ds(sid * STRIPE + k * ZB, ZB)])
            plsc.subcore_barrier()


@functools.cache
def _sc_scatter():
    return pl.kernel(
        _sc_scatter_body,
        mesh=plsc.VectorSubcoreMesh(core_axis_name="c", subcore_axis_name="s"),
        out_type=jax.ShapeDtypeStruct((4 * T,), jnp.float32),
        scratch_types=[
            pltpu.VMEM((CHUNK,), jnp.int32),
            pltpu.VMEM((CHUNK,), jnp.int32),
            pltpu.VMEM((CHUNK,), jnp.int32),
            pltpu.VMEM((CHUNK,), jnp.int32),
            pltpu.VMEM((CHUNK,), jnp.float32),
            pltpu.VMEM((CHUNK,), jnp.float32),
            pltpu.VMEM((ZB,), jnp.float32),
            pltpu.VMEM((ZB,), jnp.float32),
            pltpu.VMEM_SHARED((T,), jnp.float32),
            pltpu.SemaphoreType.DMA,
            pltpu.SemaphoreType.DMA,
            pltpu.SemaphoreType.DMA,
            pltpu.SemaphoreType.DMA,
            pltpu.SemaphoreType.DMA,
            pltpu.SemaphoreType.DMA,
        ],
    )


def _reduce_kernel(tab_ref, out_ref):
    eps = jnp.float32(1e-5)
    acc = jnp.float32(0.0)
    for s in range(4):
        b, c = s // 2, s % 2
        inter = tab_ref[4 * b + c : 4 * b + c + 1, :]
        denom = tab_ref[4 * b + 2 + c : 4 * b + 3 + c, :]
        valid = (denom > 0).astype(jnp.float32)
        dice = (2.0 * inter + eps) / (denom + eps)
        num = jnp.sum(dice * valid)
        nval = jnp.sum(valid)
        acc = acc + num / jnp.maximum(nval, 1.0)
    out_ref[:, :] = jnp.broadcast_to(1.0 - acc * 0.25, (1, 1))


_reduce_call = pl.pallas_call(
    _reduce_kernel,
    out_shape=jax.ShapeDtypeStruct((1, 1), jnp.float32),
)


def kernel(y_pred, y):
    y2 = y[:, 0]
    yp2 = y_pred[:, 1:]
    ids, vals = _prop_call(y2, yp2)
    tab = _sc_scatter()(ids.reshape(NIDS), vals.reshape(NENT))
    out = _reduce_call(tab.reshape(8, NSEG_PAD))
    return out[0, 0]


# speedup vs baseline: 1.4159x; 1.0306x over previous
import functools

import jax
import jax.numpy as jnp
from jax import lax
from jax.experimental import pallas as pl
from jax.experimental.pallas import tpu as pltpu
from jax.experimental.pallas import tpu_sc as plsc

H = 64
W = 64
D = 64
L = 2 * D
V = H * W * D
PK = W * L
NSEG_PAD = 262400
T = 2 * NSEG_PAD
STRIPE = T // 16
ZB = STRIPE // 2
NIDS = 2 * 2 * V
NENT = 2 * NIDS
IPT = NIDS // 32
CHUNK = 8192
NCHUNK = IPT // CHUNK


def _prop_kernel(y_ref, yp_ref, ids_ref, vals_ref):
    yk = y_ref[0]
    ypair = jnp.concatenate([yk, yk], axis=-1)
    li = lax.broadcasted_iota(jnp.int32, (H, W, L), 2)
    cvec = 1 + (li >= D).astype(jnp.int32)
    mask = ypair == cvec
    hi = lax.broadcasted_iota(jnp.int32, (H, W, L), 0)
    wi = lax.broadcasted_iota(jnp.int32, (H, W, L), 1)
    di = li % D
    lin = hi * (W * D) + wi * D + di + 1
    zero = jnp.zeros((H, W, L), jnp.int32)
    not_d0 = di != 0
    not_d63 = di != (D - 1)

    def mneigh(x):
        zh = jnp.zeros((1, W, L), x.dtype)
        m = jnp.maximum(x, jnp.concatenate([zh, x[:-1]], axis=0))
        m = jnp.maximum(m, jnp.concatenate([x[1:], zh], axis=0))
        zw = jnp.zeros((H, 1, L), x.dtype)
        m = jnp.maximum(m, jnp.concatenate([zw, x[:, :-1]], axis=1))
        m = jnp.maximum(m, jnp.concatenate([x[:, 1:], zw], axis=1))
        m = jnp.maximum(m, jnp.where(not_d0, pltpu.roll(x, 1, 2), zero))
        m = jnp.maximum(m, jnp.where(not_d63, pltpu.roll(x, L - 1, 2), zero))
        return m

    labels = jnp.where(mask, lin, 0)
    labels = lax.fori_loop(
        0, 8, lambda i, l: jnp.where(mask, mneigh(l), l), labels)
    vor = lax.fori_loop(
        0, 8, lambda i, v: jnp.where(v > 0, v, mneigh(v)), labels)

    ypp = jnp.concatenate([yp_ref[0, 0], yp_ref[0, 1]], axis=-1)
    p = jnp.maximum(jax.nn.sigmoid(ypp), 1e-20)
    g = mask.astype(jnp.float32)
    ids_ref[0] = vor + (cvec - 1) * NSEG_PAD
    vals_ref[0, 0] = p * g
    vals_ref[0, 1] = p + g


_prop_call = pl.pallas_call(
    _prop_kernel,
    grid=(2,),
    in_specs=[
        pl.BlockSpec((1, H, W, D), lambda b: (b, 0, 0, 0)),
        pl.BlockSpec((1, 2, H, W, D), lambda b: (b, 0, 0, 0, 0)),
    ],
    out_specs=[
        pl.BlockSpec((1, H, W, L), lambda b: (b, 0, 0, 0)),
        pl.BlockSpec((1, 2, H, W, L), lambda b: (b, 0, 0, 0, 0)),
    ],
    out_shape=[
        jax.ShapeDtypeStruct((2, H, W, L), jnp.int32),
        jax.ShapeDtypeStruct((2, 2, H, W, L), jnp.float32),
    ],
)


def _sc_scatter_body(ids_hbm, vals_hbm, out_hbm,
                     idsb0, idsb1, idsb2, idsb3, valsv0, valsv1,
                     zbuf, obuf, table,
                     isem0, isem1, isem2, isem3, vsem0, vsem1):
    cid = lax.axis_index("c")
    sid = lax.axis_index("s")
    idsbufs = (idsb0, idsb1, idsb2, idsb3)
    vbufs = ((valsv0, vsem0), (valsv1, vsem1))

    ids_base = cid * (NIDS // 2) + sid * IPT
    ih = []
    for k, (ib, isem) in enumerate(zip(idsbufs, (isem0, isem1, isem2, isem3))):
        h = pltpu.make_async_copy(
            ids_hbm.at[pl.ds(ids_base + k * CHUNK, CHUNK)], ib, isem)
        h.start()
        ih.append(h)

    def vals_off(phase, k):
        return (cid * (NENT // 2) + phase * (NIDS // 2) + sid * IPT + k * CHUNK)

    def start_vload(e0, vb):
        valsb, s = vb
        h = pltpu.make_async_copy(vals_hbm.at[pl.ds(e0, CHUNK)], valsb, s)
        h.start()
        return h

    vpend = start_vload(vals_off(0, 0), vbufs[0])

    def zfill(i, carry):
        zbuf[pl.ds(i * 16, 16)] = jnp.zeros((16,), jnp.float32)
        return carry

    lax.fori_loop(0, ZB // 16, zfill, 0)
    for k in range(2):
        pltpu.sync_copy(zbuf, table.at[pl.ds(sid * STRIPE + k * ZB, ZB)])
    plsc.subcore_barrier()

    for phase in range(2):
        for k in range(NCHUNK):
            valsb, _ = vbufs[k % 2]
            vpend.wait()
            if k + 1 < NCHUNK:
                vpend = start_vload(vals_off(phase, k + 1), vbufs[(k + 1) % 2])
            elif phase == 0:
                vpend = start_vload(vals_off(1, 0), vbufs[0])
            if phase == 0:
                ih[k].wait()
            pltpu.sync_copy(valsb, table.at[idsbufs[k]], add=True)
        plsc.subcore_barrier()
        for k in range(2):
            off = sid * STRIPE + k * ZB
            pltpu.sync_copy(table.at[pl.ds(off, ZB)], obuf)
            pltpu.sync_copy(
                obuf, out_hbm.at[pl.ds(phase * (2 * T) + cid * T + off, ZB)])
        if phase == 0:
            for k in range(2):
                pltpu.sync_copy(
                    zbuf, table.at[pl.ds(sid * STRIPE + k * ZB, ZB)])
            plsc.subcore_barrier()


@functools.cache
def _sc_scatter():
    return pl.kernel(
        _sc_scatter_body,
        mesh=plsc.VectorSubcoreMesh(core_axis_name="c", subcore_axis_name="s"),
        out_type=jax.ShapeDtypeStruct((4 * T,), jnp.float32),
        scratch_types=[
            pltpu.VMEM((CHUNK,), jnp.int32),
            pltpu.VMEM((CHUNK,), jnp.int32),
            pltpu.VMEM((CHUNK,), jnp.int32),
            pltpu.VMEM((CHUNK,), jnp.int32),
            pltpu.VMEM((CHUNK,), jnp.float32),
            pltpu.VMEM((CHUNK,), jnp.float32),
            pltpu.VMEM((ZB,), jnp.float32),
            pltpu.VMEM((ZB,), jnp.float32),
            pltpu.VMEM_SHARED((T,), jnp.float32),
            pltpu.SemaphoreType.DMA,
            pltpu.SemaphoreType.DMA,
            pltpu.SemaphoreType.DMA,
            pltpu.SemaphoreType.DMA,
            pltpu.SemaphoreType.DMA,
            pltpu.SemaphoreType.DMA,
        ],
    )


def _reduce_kernel(tab_ref, out_ref):
    eps = jnp.float32(1e-5)
    inter = tab_ref[0:4, :]
    denom = tab_ref[4:8, :]
    valid = (denom > 0).astype(jnp.float32)
    dice = (2.0 * inter + eps) / (denom + eps)
    num = jnp.sum(dice * valid, axis=1, keepdims=True)
    nval = jnp.sum(valid, axis=1, keepdims=True)
    per_bc = num / jnp.maximum(nval, 1.0)
    out_ref[:, :] = jnp.broadcast_to(1.0 - jnp.sum(per_bc) * 0.25, (1, 1))


_reduce_call = pl.pallas_call(
    _reduce_kernel,
    out_shape=jax.ShapeDtypeStruct((1, 1), jnp.float32),
)


def kernel(y_pred, y):
    y2 = y[:, 0]
    yp2 = y_pred[:, 1:]
    ids, vals = _prop_call(y2, yp2)
    tab = _sc_scatter()(ids.reshape(NIDS), vals.reshape(NENT))
    out = _reduce_call(tab.reshape(8, NSEG_PAD))
    return out[0, 0]


# speedup vs baseline: 1.5311x; 1.0813x over previous
import functools

import jax
import jax.numpy as jnp
from jax import lax
from jax.experimental import pallas as pl
from jax.experimental.pallas import tpu as pltpu
from jax.experimental.pallas import tpu_sc as plsc

H = 64
W = 64
D = 64
L = 2 * D
V = H * W * D
PK = W * L
NSEG_PAD = 262400
T = 2 * NSEG_PAD
STRIPE = T // 16
ZB = STRIPE // 2
NIDS = 2 * 2 * V
NENT = 2 * NIDS
IPT = NIDS // 32
CHUNK = 8192
NCHUNK = IPT // CHUNK


def _prop_kernel(y_ref, yp_ref, ids_ref, vals_ref):
    yk = y_ref[0]
    ypair = jnp.concatenate([yk, yk], axis=-1)
    li = lax.broadcasted_iota(jnp.int32, (H, W, L), 2)
    cvec = 1 + (li >= D).astype(jnp.int32)
    mask = ypair == cvec
    hi = lax.broadcasted_iota(jnp.int32, (H, W, L), 0)
    wi = lax.broadcasted_iota(jnp.int32, (H, W, L), 1)
    di = li % D
    lin = hi * (W * D) + wi * D + di + 1
    zero = jnp.zeros((H, W, L), jnp.int32)
    not_d0 = di != 0
    not_d63 = di != (D - 1)

    def mneigh(x):
        zh = jnp.zeros((1, W, L), x.dtype)
        m = jnp.maximum(x, jnp.concatenate([zh, x[:-1]], axis=0))
        m = jnp.maximum(m, jnp.concatenate([x[1:], zh], axis=0))
        zw = jnp.zeros((H, 1, L), x.dtype)
        m = jnp.maximum(m, jnp.concatenate([zw, x[:, :-1]], axis=1))
        m = jnp.maximum(m, jnp.concatenate([x[:, 1:], zw], axis=1))
        m = jnp.maximum(m, jnp.where(not_d0, pltpu.roll(x, 1, 2), zero))
        m = jnp.maximum(m, jnp.where(not_d63, pltpu.roll(x, L - 1, 2), zero))
        return m

    def run(step, x0):
        def cond(st):
            return (st[0] < 8) & st[2]

        def body(st):
            i, x, _ = st
            nx = step(x)
            return (i + 1, nx, jnp.any(nx != x))

        return lax.while_loop(cond, body, (0, x0, True))[1]

    labels = jnp.where(mask, lin, 0)
    labels = run(lambda l: jnp.where(mask, mneigh(l), l), labels)
    vor = run(lambda v: jnp.where(v > 0, v, mneigh(v)), labels)

    ypp = jnp.concatenate([yp_ref[0, 0], yp_ref[0, 1]], axis=-1)
    p = jnp.maximum(jax.nn.sigmoid(ypp), 1e-20)
    g = mask.astype(jnp.float32)
    ids_ref[0] = vor + (cvec - 1) * NSEG_PAD
    vals_ref[0, 0] = p * g
    vals_ref[0, 1] = p + g


_prop_call = pl.pallas_call(
    _prop_kernel,
    grid=(2,),
    in_specs=[
        pl.BlockSpec((1, H, W, D), lambda b: (b, 0, 0, 0)),
        pl.BlockSpec((1, 2, H, W, D), lambda b: (b, 0, 0, 0, 0)),
    ],
    out_specs=[
        pl.BlockSpec((1, H, W, L), lambda b: (b, 0, 0, 0)),
        pl.BlockSpec((1, 2, H, W, L), lambda b: (b, 0, 0, 0, 0)),
    ],
    out_shape=[
        jax.ShapeDtypeStruct((2, H, W, L), jnp.int32),
        jax.ShapeDtypeStruct((2, 2, H, W, L), jnp.float32),
    ],
)


def _sc_scatter_body(ids_hbm, vals_hbm, out_hbm,
                     idsb0, idsb1, idsb2, idsb3, valsv0, valsv1,
                     zbuf, obuf, table,
                     isem0, isem1, isem2, isem3, vsem0, vsem1):
    cid = lax.axis_index("c")
    sid = lax.axis_index("s")
    idsbufs = (idsb0, idsb1, idsb2, idsb3)
    vbufs = ((valsv0, vsem0), (valsv1, vsem1))

    ids_base = cid * (NIDS // 2) + sid * IPT
    ih = []
    for k, (ib, isem) in enumerate(zip(idsbufs, (isem0, isem1, isem2, isem3))):
        h = pltpu.make_async_copy(
            ids_hbm.at[pl.ds(ids_base + k * CHUNK, CHUNK)], ib, isem)
        h.start()
        ih.append(h)

    def vals_off(phase, k):
        return (cid * (NENT // 2) + phase * (NIDS // 2) + sid * IPT + k * CHUNK)

    def start_vload(e0, vb):
        valsb, s = vb
        h = pltpu.make_async_copy(vals_hbm.at[pl.ds(e0, CHUNK)], valsb, s)
        h.start()
        return h

    vpend = start_vload(vals_off(0, 0), vbufs[0])

    def zfill(i, carry):
        zbuf[pl.ds(i * 16, 16)] = jnp.zeros((16,), jnp.float32)
        return carry

    lax.fori_loop(0, ZB // 16, zfill, 0)
    for k in range(2):
        pltpu.sync_copy(zbuf, table.at[pl.ds(sid * STRIPE + k * ZB, ZB)])
    plsc.subcore_barrier()

    for phase in range(2):
        for k in range(NCHUNK):
            valsb, _ = vbufs[k % 2]
            vpend.wait()
            if k + 1 < NCHUNK:
                vpend = start_vload(vals_off(phase, k + 1), vbufs[(k + 1) % 2])
            elif phase == 0:
                vpend = start_vload(vals_off(1, 0), vbufs[0])
            if phase == 0:
                ih[k].wait()
            pltpu.sync_copy(valsb, table.at[idsbufs[k]], add=True)
        plsc.subcore_barrier()
        for k in range(2):
            off = sid * STRIPE + k * ZB
            pltpu.sync_copy(table.at[pl.ds(off, ZB)], obuf)
            pltpu.sync_copy(
                obuf, out_hbm.at[pl.ds(phase * (2 * T) + cid * T + off, ZB)])
        if phase == 0:
            for k in range(2):
                pltpu.sync_copy(
                    zbuf, table.at[pl.ds(sid * STRIPE + k * ZB, ZB)])
            plsc.subcore_barrier()


@functools.cache
def _sc_scatter():
    return pl.kernel(
        _sc_scatter_body,
        mesh=plsc.VectorSubcoreMesh(core_axis_name="c", subcore_axis_name="s"),
        out_type=jax.ShapeDtypeStruct((4 * T,), jnp.float32),
        scratch_types=[
            pltpu.VMEM((CHUNK,), jnp.int32),
            pltpu.VMEM((CHUNK,), jnp.int32),
            pltpu.VMEM((CHUNK,), jnp.int32),
            pltpu.VMEM((CHUNK,), jnp.int32),
            pltpu.VMEM((CHUNK,), jnp.float32),
            pltpu.VMEM((CHUNK,), jnp.float32),
            pltpu.VMEM((ZB,), jnp.float32),
            pltpu.VMEM((ZB,), jnp.float32),
            pltpu.VMEM_SHARED((T,), jnp.float32),
            pltpu.SemaphoreType.DMA,
            pltpu.SemaphoreType.DMA,
            pltpu.SemaphoreType.DMA,
            pltpu.SemaphoreType.DMA,
            pltpu.SemaphoreType.DMA,
            pltpu.SemaphoreType.DMA,
        ],
    )


def _reduce_kernel(tab_ref, out_ref):
    eps = jnp.float32(1e-5)
    inter = tab_ref[0:4, :]
    denom = tab_ref[4:8, :]
    valid = (denom > 0).astype(jnp.float32)
    dice = (2.0 * inter + eps) / (denom + eps)
    num = jnp.sum(dice * valid, axis=1, keepdims=True)
    nval = jnp.sum(valid, axis=1, keepdims=True)
    per_bc = num / jnp.maximum(nval, 1.0)
    out_ref[:, :] = jnp.broadcast_to(1.0 - jnp.sum(per_bc) * 0.25, (1, 1))


_reduce_call = pl.pallas_call(
    _reduce_kernel,
    out_shape=jax.ShapeDtypeStruct((1, 1), jnp.float32),
)


def kernel(y_pred, y):
    y2 = y[:, 0]
    yp2 = y_pred[:, 1:]
    ids, vals = _prop_call(y2, yp2)
    tab = _sc_scatter()(ids.reshape(NIDS), vals.reshape(NENT))
    out = _reduce_call(tab.reshape(8, NSEG_PAD))
    return out[0, 0]
